# baseline (speedup 1.0000x reference)
import functools

import jax
import jax.numpy as jnp
from jax import lax
from jax.experimental import pallas as pl
from jax.experimental.pallas import tpu as pltpu
from jax.experimental.pallas import tpu_sc as plsc

_BS = 1024
_CIN = 2
_HW = 1024
_ZD = 64
_NCH = 3
_LANES = 16
_LTILE = 128
_WAVE = 4
_NBUF = 2


def _sc_gather(idx_i32, table_t):
    info = plsc.get_sparse_core_info()
    num_cores = info.num_cores
    nw = num_cores * info.num_subcores
    bpw = _BS // nw
    mesh = plsc.VectorSubcoreMesh(core_axis_name="c", subcore_axis_name="s")

    @functools.partial(
        pl.kernel,
        mesh=mesh,
        out_type=jax.ShapeDtypeStruct((_BS, _ZD), jnp.float32),
        compiler_params=pltpu.CompilerParams(needs_layout_passes=False),
        scratch_types=[
            pltpu.VMEM((bpw,), jnp.int32),
            pltpu.VMEM((_NBUF, _WAVE, _ZD, _LTILE), jnp.float32),
            pltpu.VMEM((bpw, _ZD), jnp.float32),
            pltpu.SemaphoreType.DMA,
        ],
    )
    def gather_kernel(idx_hbm, table_hbm, out_hbm, idx_v, slab_v, rows_v,
                      sem):
        wid = lax.axis_index("s") * num_cores + lax.axis_index("c")
        base = wid * bpw
        pltpu.sync_copy(idx_hbm.at[pl.ds(base, bpw)], idx_v)
        lane = lax.broadcasted_iota(jnp.int32, (_LANES,), 0)
        zlane = lax.broadcasted_iota(jnp.int32, (_LANES,), 0)
        n_waves = bpw // _WAVE

        def fire(wave):
            entries = []
            for s in range(_WAVE):
                j = wave * _WAVE + s
                vals = idx_v[pl.ds((j // _LANES) * _LANES, _LANES)]
                r = jnp.max(jnp.where(lane == (j % _LANES), vals, 0))
                cp = pltpu.async_copy(
                    table_hbm.at[:, pl.ds((r // _LTILE) * _LTILE, _LTILE)],
                    slab_v.at[wave % _NBUF, s],
                    sem,
                )
                entries.append((r, cp))
            return entries

        def extract(wave, entries):
            for s in range(_WAVE):
                j = wave * _WAVE + s
                r, _ = entries[s]
                bvec = jnp.full((_LANES,), wave % _NBUF, jnp.int32)
                svec = jnp.full((_LANES,), s, jnp.int32)
                lvec = jnp.full((_LANES,), r % _LTILE, jnp.int32)
                for k in range(_ZD // _LANES):
                    feats = plsc.load_gather(
                        slab_v, [bvec, svec, zlane + k * _LANES, lvec])
                    rows_v[j, pl.ds(k * _LANES, _LANES)] = feats

        prev = fire(0)
        for wave in range(1, n_waves):
            cur = fire(wave)
            for _, cp in prev:
                cp.wait()
            extract(wave - 1, prev)
            prev = cur
        for _, cp in prev:
            cp.wait()
        extract(n_waves - 1, prev)
        pltpu.sync_copy(rows_v, out_hbm.at[pl.ds(base, bpw)])

    return gather_kernel(idx_i32, table_t)


def _local_body(lc_ref, ws_ref, b_ref, out_ref):
    lc0 = lc_ref[0, :, :]
    lc1 = lc_ref[1, :, :]
    for ch in range(_NCH):
        out_ref[ch, :, :] = (
            lc0 * ws_ref[ch, 0] + lc1 * ws_ref[ch, 1] + b_ref[ch]
        )


def _local(lc_t, w_t, b_syn, block_hw=512):
    grid = (_HW // block_hw,)
    return pl.pallas_call(
        _local_body,
        grid=grid,
        in_specs=[
            pl.BlockSpec((_CIN, block_hw, _BS), lambda i: (0, i, 0)),
            pl.BlockSpec(memory_space=pltpu.SMEM),
            pl.BlockSpec(memory_space=pltpu.SMEM),
        ],
        out_specs=pl.BlockSpec((_NCH, block_hw, _BS), lambda i: (0, i, 0)),
        out_shape=jax.ShapeDtypeStruct((_NCH, _HW, _BS), jnp.float32),
    )(lc_t, w_t, b_syn)


def _addv_body(a_ref, lat_ref, wt_ref, out_ref):
    v = lax.dot_general(
        wt_ref[:, 2:], lat_ref[...],
        dimension_numbers=(((1,), (1,)), ((), ())),
        preferred_element_type=jnp.float32,
    )
    for ch in range(_NCH):
        out_ref[ch, :, :] = a_ref[ch, :, :] + v[ch, :][None, :]


def _addv(acc, lat, w_t, block_hw=512):
    grid = (_HW // block_hw,)
    return pl.pallas_call(
        _addv_body,
        grid=grid,
        in_specs=[
            pl.BlockSpec((_NCH, block_hw, _BS), lambda i: (0, i, 0)),
            pl.BlockSpec((_BS, _ZD), lambda i: (0, 0)),
            pl.BlockSpec((_NCH, _CIN + _ZD), lambda i: (0, 0)),
        ],
        out_specs=pl.BlockSpec((_NCH, block_hw, _BS), lambda i: (0, i, 0)),
        out_shape=jax.ShapeDtypeStruct((_NCH, _HW, _BS), jnp.float32),
        input_output_aliases={0: 0},
    )(acc, lat, w_t)


def kernel(local_coords, idx, latent_codes, W_syn, b_syn):
    bs, c, h, w = local_coords.shape
    lc_t = jnp.transpose(local_coords, (1, 2, 3, 0)).reshape(c, h * w, bs)
    table_t = jnp.transpose(latent_codes, (1, 0))
    w_t = jnp.transpose(W_syn, (1, 0))
    lat = _sc_gather(idx.astype(jnp.int32), table_t)
    acc = _local(lc_t, w_t, b_syn)
    out_t = _addv(acc, lat, w_t)
    return jnp.transpose(out_t.reshape(_NCH, h, w, bs), (3, 0, 1, 2))


# speedup vs baseline: 1.0574x; 1.0574x over previous
import functools

import jax
import jax.numpy as jnp
from jax import lax
from jax.experimental import pallas as pl
from jax.experimental.pallas import tpu as pltpu
from jax.experimental.pallas import tpu_sc as plsc

_BS = 1024
_CIN = 2
_HW = 1024
_ZD = 64
_NCH = 3
_LANES = 16
_LTILE = 128
_WAVE = 4
_NBUF = 2


def _sc_gather(idx_i32, table_t):
    info = plsc.get_sparse_core_info()
    num_cores = info.num_cores
    nw = num_cores * info.num_subcores
    bpw = _BS // nw
    mesh = plsc.VectorSubcoreMesh(core_axis_name="c", subcore_axis_name="s")

    @functools.partial(
        pl.kernel,
        mesh=mesh,
        out_type=jax.ShapeDtypeStruct((_BS, _ZD), jnp.float32),
        compiler_params=pltpu.CompilerParams(needs_layout_passes=False),
        scratch_types=[
            pltpu.VMEM((bpw,), jnp.int32),
            pltpu.VMEM((_NBUF, _WAVE, _ZD, _LTILE), jnp.float32),
            pltpu.VMEM((bpw, _ZD), jnp.float32),
            pltpu.SemaphoreType.DMA,
        ],
    )
    def gather_kernel(idx_hbm, table_hbm, out_hbm, idx_v, slab_v, rows_v,
                      sem):
        wid = lax.axis_index("s") * num_cores + lax.axis_index("c")
        base = wid * bpw
        pltpu.sync_copy(idx_hbm.at[pl.ds(base, bpw)], idx_v)
        lane = lax.broadcasted_iota(jnp.int32, (_LANES,), 0)
        zlane = lax.broadcasted_iota(jnp.int32, (_LANES,), 0)
        n_waves = bpw // _WAVE

        def fire(wave):
            entries = []
            for s in range(_WAVE):
                j = wave * _WAVE + s
                vals = idx_v[pl.ds((j // _LANES) * _LANES, _LANES)]
                r = jnp.max(jnp.where(lane == (j % _LANES), vals, 0))
                cp = pltpu.async_copy(
                    table_hbm.at[:, pl.ds((r // _LTILE) * _LTILE, _LTILE)],
                    slab_v.at[wave % _NBUF, s],
                    sem,
                )
                entries.append((r, cp))
            return entries

        def extract(wave, entries):
            for s in range(_WAVE):
                j = wave * _WAVE + s
                r, _ = entries[s]
                bvec = jnp.full((_LANES,), wave % _NBUF, jnp.int32)
                svec = jnp.full((_LANES,), s, jnp.int32)
                lvec = jnp.full((_LANES,), r % _LTILE, jnp.int32)
                for k in range(_ZD // _LANES):
                    feats = plsc.load_gather(
                        slab_v, [bvec, svec, zlane + k * _LANES, lvec])
                    rows_v[j, pl.ds(k * _LANES, _LANES)] = feats

        prev = fire(0)
        for wave in range(1, n_waves):
            cur = fire(wave)
            for _, cp in prev:
                cp.wait()
            extract(wave - 1, prev)
            prev = cur
        for _, cp in prev:
            cp.wait()
        extract(n_waves - 1, prev)
        pltpu.sync_copy(rows_v, out_hbm.at[pl.ds(base, bpw)])

    return gather_kernel(idx_i32, table_t)


def _dense_body(lc_ref, lat_ref, wt_ref, ws_ref, b_ref, out_ref):
    v = lax.dot_general(
        wt_ref[:, 2:], lat_ref[...],
        dimension_numbers=(((1,), (1,)), ((), ())),
        preferred_element_type=jnp.float32,
    )
    lc0 = lc_ref[0, :, :]
    lc1 = lc_ref[1, :, :]
    for ch in range(_NCH):
        out_ref[ch, :, :] = (
            lc0 * ws_ref[ch, 0]
            + lc1 * ws_ref[ch, 1]
            + (v[ch, :] + b_ref[ch])[None, :]
        )


def _dense(lc_t, lat, w_t, b_syn, block_hw=512):
    grid = (_HW // block_hw,)
    return pl.pallas_call(
        _dense_body,
        grid=grid,
        in_specs=[
            pl.BlockSpec((_CIN, block_hw, _BS), lambda i: (0, i, 0)),
            pl.BlockSpec((_BS, _ZD), lambda i: (0, 0)),
            pl.BlockSpec((_NCH, _CIN + _ZD), lambda i: (0, 0)),
            pl.BlockSpec(memory_space=pltpu.SMEM),
            pl.BlockSpec(memory_space=pltpu.SMEM),
        ],
        out_specs=pl.BlockSpec((_NCH, block_hw, _BS), lambda i: (0, i, 0)),
        out_shape=jax.ShapeDtypeStruct((_NCH, _HW, _BS), jnp.float32),
    )(lc_t, lat, w_t, w_t, b_syn)


def kernel(local_coords, idx, latent_codes, W_syn, b_syn):
    bs, c, h, w = local_coords.shape
    lc_t = jnp.transpose(local_coords, (1, 2, 3, 0)).reshape(c, h * w, bs)
    table_t = jnp.transpose(latent_codes, (1, 0))
    lat = _sc_gather(idx.astype(jnp.int32), table_t)
    w_t = jnp.transpose(W_syn, (1, 0))
    out_t = _dense(lc_t, lat, w_t, b_syn)
    return jnp.transpose(out_t.reshape(_NCH, h, w, bs), (3, 0, 1, 2))
